# Initial kernel scaffold; baseline (speedup 1.0000x reference)
import jax
import jax.numpy as jnp
from jax.experimental import pallas as pl


def kernel(state, chunk_embeddings, top_k, W1, b1, W2, b2):
    raise NotImplementedError("write your pallas kernel here")


# speedup vs baseline: 5.3818x; 5.3818x over previous
import functools

import jax
import jax.numpy as jnp
from jax import lax
from jax.experimental import pallas as pl
from jax.experimental.pallas import tpu as pltpu
from jax.experimental.pallas import tpu_sc as plsc

_NEG = -3e38
_IBIG = 2**30
_K = 64
_G = 16


def _scores_body(N, blk_n, q_ref, c_ref, s_ref, gm_ref):
    j = pl.program_id(1)
    s = lax.dot_general(q_ref[...], c_ref[...], (((1,), (1,)), ((), ())),
                        preferred_element_type=jnp.float32)
    col = j * blk_n + lax.broadcasted_iota(jnp.int32, s.shape, 1)
    s = jnp.where(col < N, s, _NEG)
    blk_b = s.shape[0]
    s_ref[...] = s.reshape(blk_b, blk_n // 128, 128)
    gm_ref[...] = jnp.max(s.reshape(blk_b, blk_n // _G, _G), axis=2)


def _make_scores(B, N, d_chunk, blk_b=256, blk_n=2048):
    n_tiles = pl.cdiv(N, blk_n)
    n128 = n_tiles * blk_n // 128
    return pl.pallas_call(
        functools.partial(_scores_body, N, blk_n),
        grid=(B // blk_b, n_tiles),
        in_specs=[
            pl.BlockSpec((blk_b, d_chunk), lambda i, j: (i, 0)),
            pl.BlockSpec((blk_n, d_chunk), lambda i, j: (j, 0)),
        ],
        out_specs=[
            pl.BlockSpec((blk_b, blk_n // 128, 128), lambda i, j: (i, j, 0)),
            pl.BlockSpec((blk_b, blk_n // _G), lambda i, j: (i, j)),
        ],
        out_shape=[
            jax.ShapeDtypeStruct((B, n128, 128), jnp.float32),
            jax.ShapeDtypeStruct((B, n_tiles * (blk_n // _G)), jnp.float32),
        ],
    )


def _groups_body(n128, blk_b, gw, gm_ref, gid_ref, gblk_ref, gmod_ref):
    i = pl.program_id(0)
    v = gm_ref[...]
    gidx = lax.broadcasted_iota(jnp.int32, (blk_b, gw), 1)
    rbase = (i * blk_b + lax.broadcasted_iota(jnp.int32, (blk_b, 1), 0)) * n128
    gpb = 128 // _G
    for t in range(_K):
        m = jnp.max(v, axis=1, keepdims=True)
        g = jnp.min(jnp.where(v == m, gidx, _IBIG), axis=1, keepdims=True)
        gid_ref[:, t:t + 1] = g
        gblk_ref[:, t:t + 1] = rbase + g // gpb
        gmod_ref[:, t:t + 1] = g % gpb
        v = jnp.where(gidx == g, _NEG, v)


def _make_groups(B, gw, n128, blk_b=256):
    return pl.pallas_call(
        functools.partial(_groups_body, n128, blk_b, gw),
        grid=(B // blk_b,),
        in_specs=[pl.BlockSpec((blk_b, gw), lambda i: (i, 0))],
        out_specs=[
            pl.BlockSpec((blk_b, _K), lambda i: (i, 0)),
            pl.BlockSpec((blk_b, _K), lambda i: (i, 0)),
            pl.BlockSpec((blk_b, _K), lambda i: (i, 0)),
        ],
        out_shape=[
            jax.ShapeDtypeStruct((B, _K), jnp.int32),
            jax.ShapeDtypeStruct((B, _K), jnp.int32),
            jax.ShapeDtypeStruct((B, _K), jnp.int32),
        ],
    )


def _make_sc_gather(B):
    info = plsc.get_sparse_core_info()
    nw = info.num_cores * info.num_subcores
    rows_per_w = B // nw
    mesh = plsc.VectorSubcoreMesh(core_axis_name="c", subcore_axis_name="s")

    @functools.partial(
        pl.kernel,
        mesh=mesh,
        out_type=jax.ShapeDtypeStruct((B, _K, 128), jnp.float32),
        scratch_types=[
            pltpu.VMEM((_K,), jnp.int32),
            pltpu.VMEM((_K, 128), jnp.float32),
            pltpu.SemaphoreType.DMA,
        ],
    )
    def k(scores128_hbm, gblk_hbm, cand_hbm, idx_v, rows_v, sem):
        wid = lax.axis_index("s") * info.num_cores + lax.axis_index("c")
        base = wid * rows_per_w

        def body(r, carry):
            pltpu.sync_copy(gblk_hbm.at[base + r], idx_v)
            pltpu.async_copy(scores128_hbm.at[idx_v], rows_v, sem).wait()
            pltpu.sync_copy(rows_v, cand_hbm.at[base + r])
            return carry

        lax.fori_loop(0, rows_per_w, body, 0)

    return k


def _topk_body(blk_b, cand_ref, gid_ref, gmod_ref, oi_ref, ov_ref, idx_ref,
               v_ref):
    gpb = 128 // _G
    lane16 = lax.broadcasted_iota(jnp.int32, (blk_b, _G), 1)
    for a in range(_K):
        ga = gmod_ref[:, a:a + 1]
        acc = cand_ref[:, a, 0:_G]
        for w in range(1, gpb):
            acc = jnp.where(ga == w, cand_ref[:, a, w * _G:(w + 1) * _G], acc)
        v_ref[:, a * _G:(a + 1) * _G] = acc
        idx_ref[:, a * _G:(a + 1) * _G] = gid_ref[:, a:a + 1] * _G + lane16
    v = v_ref[...]
    cidx = idx_ref[...]
    for t in range(_K):
        m = jnp.max(v, axis=1, keepdims=True)
        j = jnp.min(jnp.where(v == m, cidx, _IBIG), axis=1, keepdims=True)
        ov_ref[:, t:t + 1] = m
        oi_ref[:, t:t + 1] = j
        v = jnp.where(cidx == j, _NEG, v)


def _make_topk(B, blk_b=256):
    cw = _K * _G
    return pl.pallas_call(
        functools.partial(_topk_body, blk_b),
        grid=(B // blk_b,),
        in_specs=[
            pl.BlockSpec((blk_b, _K, 128), lambda i: (i, 0, 0)),
            pl.BlockSpec((blk_b, _K), lambda i: (i, 0)),
            pl.BlockSpec((blk_b, _K), lambda i: (i, 0)),
        ],
        out_specs=[
            pl.BlockSpec((blk_b, _K), lambda i: (i, 0)),
            pl.BlockSpec((blk_b, _K), lambda i: (i, 0)),
        ],
        out_shape=[
            jax.ShapeDtypeStruct((B, _K), jnp.int32),
            jax.ShapeDtypeStruct((B, _K), jnp.float32),
        ],
        scratch_shapes=[pltpu.VMEM((blk_b, cw), jnp.int32),
                        pltpu.VMEM((blk_b, cw), jnp.float32)],
    )


def kernel(state, chunk_embeddings, top_k, W1, b1, W2, b2):
    B, d_state = state.shape
    N, d_chunk = chunk_embeddings.shape
    n_groups = N // _G

    h = jnp.maximum(state @ W1.T + b1, 0.0)
    qraw = h @ W2.T + b2
    q = qraw / jnp.maximum(jnp.linalg.norm(qraw, axis=-1, keepdims=True), 1e-12)
    c = chunk_embeddings / jnp.maximum(
        jnp.linalg.norm(chunk_embeddings, axis=-1, keepdims=True), 1e-12)

    scores3, gm = _make_scores(B, N, d_chunk)(q, c)
    n128 = scores3.shape[1]
    gid, gblk, gmod = _make_groups(B, gm.shape[1], n128)(gm)
    scores128 = scores3.reshape(B * n128, 128)
    cand = _make_sc_gather(B)(scores128, gblk)
    idx, vals = _make_topk(B)(cand, gid, gmod)
    return (idx, vals)
